# Initial kernel scaffold; baseline (speedup 1.0000x reference)
import jax
import jax.numpy as jnp
from jax.experimental import pallas as pl


def kernel(x, edge_index, edge_type, tensor_slice, W1, q1, k1, b1, W2, q2, k2, b2, Wl, bl):
    raise NotImplementedError("write your pallas kernel here")


# speedup vs baseline: 20.5146x; 20.5146x over previous
import functools

import jax
import jax.numpy as jnp
from jax import lax
from jax.experimental import pallas as pl
from jax.experimental.pallas import tpu as pltpu
from jax.experimental.pallas import tpu_sc as plsc

_BN = 512
_W = 128
_LANES = 16


def _transform(x, W, q, k):
    Np, D = x.shape
    R = W.shape[0]
    NB = Np // _BN

    def body(x_ref, w_ref, q_ref, k_ref, xw_ref, aq_ref, ak_ref):
        xw = jnp.dot(x_ref[...], w_ref[0], preferred_element_type=jnp.float32)
        xw_ref[0] = xw
        aq_ref[0] = jnp.sum(xw * q_ref[0], axis=1, keepdims=True)
        ak_ref[0] = jnp.sum(xw * k_ref[0], axis=1, keepdims=True)

    return pl.pallas_call(
        body,
        grid=(NB, R),
        in_specs=[
            pl.BlockSpec((_BN, D), lambda nb, r: (nb, 0)),
            pl.BlockSpec((1, D, D), lambda nb, r: (r, 0, 0)),
            pl.BlockSpec((1, 1, D), lambda nb, r: (r, 0, 0)),
            pl.BlockSpec((1, 1, D), lambda nb, r: (r, 0, 0)),
        ],
        out_specs=[
            pl.BlockSpec((1, _BN, D), lambda nb, r: (r, nb, 0)),
            pl.BlockSpec((1, _BN, 1), lambda nb, r: (r, nb, 0)),
            pl.BlockSpec((1, _BN, 1), lambda nb, r: (r, nb, 0)),
        ],
        out_shape=[
            jax.ShapeDtypeStruct((R, Np, D), jnp.float32),
            jax.ShapeDtypeStruct((R, Np, 1), jnp.float32),
            jax.ShapeDtypeStruct((R, Np, 1), jnp.float32),
        ],
    )(x, W, q, k)


def _edge_pass(src, dst, et, aq_flat, ak_flat, xw_flat, Np, D):
    Ep = src.shape[0]
    info = plsc.get_sparse_core_info()
    NC, NS = info.num_cores, info.num_subcores
    NW = NC * NS
    ew = Ep // NW
    nwin = ew // _W
    zrows = Np // NS
    nz = zrows // _W

    mesh = plsc.VectorSubcoreMesh(core_axis_name="c", subcore_axis_name="s")

    @functools.partial(
        pl.kernel,
        mesh=mesh,
        out_type=[
            jax.ShapeDtypeStruct((NC, Np, D), jnp.float32),
            jax.ShapeDtypeStruct((NC, Np), jnp.float32),
        ],
        scratch_types=[
            pltpu.VMEM((_W,), jnp.int32),
            pltpu.VMEM((_W,), jnp.int32),
            pltpu.VMEM((_W,), jnp.int32),
            pltpu.VMEM((_W,), jnp.int32),
            pltpu.VMEM((_W,), jnp.int32),
            pltpu.VMEM((_W,), jnp.float32),
            pltpu.VMEM((_W,), jnp.float32),
            pltpu.VMEM((_W,), jnp.float32),
            pltpu.VMEM((_W, D), jnp.float32),
            pltpu.VMEM((_W, D), jnp.float32),
            pltpu.VMEM((zrows,), jnp.float32),
            pltpu.VMEM_SHARED((Np, D), jnp.float32),
            pltpu.VMEM_SHARED((Np,), jnp.float32),
            pltpu.SemaphoreType.DMA,
        ],
    )
    def run(src_h, dst_h, et_h, aq_h, ak_h, xw_h, acc_out, den_out,
            srcb, dstb, etb, gidx, idxi, ai, aj, ee, rows, zbuf, zden,
            acc_sh, den_sh, sem):
        core = lax.axis_index("c")
        sid = lax.axis_index("s")
        wid = sid * NC + core
        zero16 = jnp.zeros((_LANES,), jnp.float32)

        def zb_body(i, _):
            for kk in range(D // _LANES):
                zbuf[i, pl.ds(kk * _LANES, _LANES)] = zero16
            return 0
        lax.fori_loop(0, _W, zb_body, 0)

        def zd_body(i, _):
            zden[pl.ds(i * _LANES, _LANES)] = zero16
            return 0
        lax.fori_loop(0, zrows // _LANES, zd_body, 0)

        for j in range(nz):
            pltpu.sync_copy(zbuf, acc_sh.at[pl.ds(sid * zrows + j * _W, _W)])
        pltpu.sync_copy(zden, den_sh.at[pl.ds(sid * zrows, zrows)])
        plsc.subcore_barrier()

        ebase = wid * ew

        def win_body(w, _):
            base = ebase + w * _W
            pltpu.sync_copy(src_h.at[pl.ds(base, _W)], srcb)
            pltpu.sync_copy(dst_h.at[pl.ds(base, _W)], dstb)
            pltpu.sync_copy(et_h.at[pl.ds(base, _W)], etb)
            for v in range(_W // _LANES):
                sl = pl.ds(v * _LANES, _LANES)
                t16 = etb[sl] * Np
                gidx[sl] = t16 + srcb[sl]
                idxi[sl] = t16 + dstb[sl]
            pltpu.async_copy(aq_h.at[idxi], ai, sem).wait()
            pltpu.async_copy(ak_h.at[gidx], aj, sem).wait()
            for v in range(_W // _LANES):
                sl = pl.ds(v * _LANES, _LANES)
                e16 = ai[sl] + aj[sl]
                e16 = jnp.where(e16 >= 0.0, e16, 0.2 * e16)
                ee[sl] = jnp.exp(e16)
            pltpu.sync_copy(ee, den_sh.at[dstb], add=True)
            pltpu.async_copy(xw_h.at[gidx], rows, sem).wait()

            def scale_grp(g, _):
                a16 = ee[pl.ds(g * _LANES, _LANES)]
                for j in range(_LANES):
                    a = a16[j]
                    i = g * _LANES + j
                    for kk in range(D // _LANES):
                        sl = pl.ds(kk * _LANES, _LANES)
                        rows[i, sl] = rows[i, sl] * a
                return 0
            lax.fori_loop(0, _W // _LANES, scale_grp, 0)
            pltpu.sync_copy(rows, acc_sh.at[dstb], add=True)
            return 0

        lax.fori_loop(0, nwin, win_body, 0)
        plsc.subcore_barrier()

        @pl.when(sid == 0)
        def _():
            pltpu.sync_copy(acc_sh, acc_out.at[core])
            pltpu.sync_copy(den_sh, den_out.at[core])

    return run(src, dst, et, aq_flat, ak_flat, xw_flat)


def _finish(acc, den3, b):
    _, Np, D = acc.shape
    NB = Np // _BN

    def body(a_ref, d_ref, b_ref, h_ref):
        a = a_ref[0] + a_ref[1]
        dd = d_ref[0] + d_ref[1] + 1e-16
        h_ref[...] = jnp.maximum(a / dd + b_ref[...], 0.0)

    return pl.pallas_call(
        body,
        grid=(NB,),
        in_specs=[
            pl.BlockSpec((2, _BN, D), lambda nb: (0, nb, 0)),
            pl.BlockSpec((2, _BN, 1), lambda nb: (0, nb, 0)),
            pl.BlockSpec((1, D), lambda nb: (0, 0)),
        ],
        out_specs=pl.BlockSpec((_BN, D), lambda nb: (nb, 0)),
        out_shape=jax.ShapeDtypeStruct((Np, D), jnp.float32),
    )(acc, den3, b)


def _head(acc, den3, b, Wl, bl):
    _, Np, D = acc.shape
    DO = Wl.shape[1]
    NB = Np // _BN

    def body(a_ref, d_ref, b_ref, wl_ref, bl_ref, o_ref):
        a = a_ref[0] + a_ref[1]
        dd = d_ref[0] + d_ref[1] + 1e-16
        h = jnp.maximum(a / dd + b_ref[...], 0.0)
        logits = jnp.dot(h, wl_ref[...], preferred_element_type=jnp.float32)
        logits = logits + bl_ref[...]
        m = jnp.max(logits, axis=1, keepdims=True)
        s = logits - m
        o_ref[...] = s - jnp.log(jnp.sum(jnp.exp(s), axis=1, keepdims=True))

    return pl.pallas_call(
        body,
        grid=(NB,),
        in_specs=[
            pl.BlockSpec((2, _BN, D), lambda nb: (0, nb, 0)),
            pl.BlockSpec((2, _BN, 1), lambda nb: (0, nb, 0)),
            pl.BlockSpec((1, D), lambda nb: (0, 0)),
            pl.BlockSpec((D, DO), lambda nb: (0, 0)),
            pl.BlockSpec((1, DO), lambda nb: (0, 0)),
        ],
        out_specs=pl.BlockSpec((_BN, DO), lambda nb: (nb, 0)),
        out_shape=jax.ShapeDtypeStruct((Np, DO), jnp.float32),
    )(acc, den3, b, Wl, bl)


def kernel(x, edge_index, edge_type, tensor_slice,
           W1, q1, k1, b1, W2, q2, k2, b2, Wl, bl):
    N, D = x.shape
    R = W1.shape[0]
    E = edge_type.shape[0]

    Np = ((N + _BN - 1) // _BN) * _BN
    NW = 32
    ew = ((E + NW * _W - 1) // (NW * _W)) * _W
    Ep = ew * NW

    xp = jnp.pad(x, ((0, Np - N), (0, 0)))
    src = edge_index[0]
    dst = edge_index[1]
    pad = Ep - E
    src_p = jnp.concatenate([src, jnp.zeros((pad,), jnp.int32)])
    dst_p = jnp.concatenate(
        [dst, Np - 8 + (jnp.arange(pad, dtype=jnp.int32) % 8)])
    et_p = jnp.concatenate([edge_type, jnp.zeros((pad,), jnp.int32)])

    q1r = q1.reshape(R, 1, D)
    k1r = k1.reshape(R, 1, D)
    q2r = q2.reshape(R, 1, D)
    k2r = k2.reshape(R, 1, D)

    xw1, aq1, ak1 = _transform(xp, W1, q1r, k1r)
    acc1, den1 = _edge_pass(src_p, dst_p, et_p,
                            aq1.reshape(R * Np), ak1.reshape(R * Np),
                            xw1.reshape(R * Np, D), Np, D)
    h = _finish(acc1, den1.reshape(2, Np, 1), b1.reshape(1, D))

    xw2, aq2, ak2 = _transform(h, W2, q2r, k2r)
    acc2, den2 = _edge_pass(src_p, dst_p, et_p,
                            aq2.reshape(R * Np), ak2.reshape(R * Np),
                            xw2.reshape(R * Np, D), Np, D)
    out = _head(acc2, den2.reshape(2, Np, 1), b2.reshape(1, D),
                Wl, bl.reshape(1, Wl.shape[1]))
    return out[:N]


# speedup vs baseline: 23.3953x; 1.1404x over previous
import functools

import jax
import jax.numpy as jnp
from jax import lax
from jax.experimental import pallas as pl
from jax.experimental.pallas import tpu as pltpu
from jax.experimental.pallas import tpu_sc as plsc

_BN = 512
_W = 128
_LANES = 16


def _transform(x, W, q, k):
    Np, D = x.shape
    R = W.shape[0]
    NB = Np // _BN

    def body(x_ref, w_ref, q_ref, k_ref, xw_ref, aq_ref, ak_ref):
        xw = jnp.dot(x_ref[...], w_ref[0], preferred_element_type=jnp.float32)
        xw_ref[0] = xw
        aq_ref[0] = jnp.sum(xw * q_ref[0], axis=1, keepdims=True)
        ak_ref[0] = jnp.sum(xw * k_ref[0], axis=1, keepdims=True)

    return pl.pallas_call(
        body,
        grid=(NB, R),
        in_specs=[
            pl.BlockSpec((_BN, D), lambda nb, r: (nb, 0)),
            pl.BlockSpec((1, D, D), lambda nb, r: (r, 0, 0)),
            pl.BlockSpec((1, 1, D), lambda nb, r: (r, 0, 0)),
            pl.BlockSpec((1, 1, D), lambda nb, r: (r, 0, 0)),
        ],
        out_specs=[
            pl.BlockSpec((1, _BN, D), lambda nb, r: (r, nb, 0)),
            pl.BlockSpec((1, _BN, 1), lambda nb, r: (r, nb, 0)),
            pl.BlockSpec((1, _BN, 1), lambda nb, r: (r, nb, 0)),
        ],
        out_shape=[
            jax.ShapeDtypeStruct((R, Np, D), jnp.float32),
            jax.ShapeDtypeStruct((R, Np, 1), jnp.float32),
            jax.ShapeDtypeStruct((R, Np, 1), jnp.float32),
        ],
    )(x, W, q, k)


def _edge_pass(edges3, aq_flat, ak_flat, xw_flat, Np, D):
    NW, nwin, _, _ = edges3.shape
    info = plsc.get_sparse_core_info()
    NC, NS = info.num_cores, info.num_subcores
    assert NW == NC * NS and nwin % 2 == 0
    zrows = Np // NS
    nz = zrows // _W

    mesh = plsc.VectorSubcoreMesh(core_axis_name="c", subcore_axis_name="s")

    @functools.partial(
        pl.kernel,
        mesh=mesh,
        out_type=[
            jax.ShapeDtypeStruct((NC, Np, D), jnp.float32),
            jax.ShapeDtypeStruct((NC, Np), jnp.float32),
        ],
        scratch_types=[
            pltpu.VMEM((3, _W), jnp.int32),
            pltpu.VMEM((2, _W), jnp.int32),
            pltpu.VMEM((2, _W), jnp.int32),
            pltpu.VMEM((2, _W), jnp.int32),
            pltpu.VMEM((_W,), jnp.float32),
            pltpu.VMEM((_W,), jnp.float32),
            pltpu.VMEM((_W,), jnp.float32),
            pltpu.VMEM((_W, D), jnp.float32),
            pltpu.VMEM((_W, D), jnp.float32),
            pltpu.VMEM((zrows,), jnp.float32),
            pltpu.VMEM_SHARED((Np, D), jnp.float32),
            pltpu.VMEM_SHARED((Np,), jnp.float32),
            pltpu.SemaphoreType.DMA,
            pltpu.SemaphoreType.DMA,
            pltpu.SemaphoreType.DMA,
        ],
    )
    def run(e3_h, aq_h, ak_h, xw_h, acc_out, den_out,
            e3buf, dstb, gidxb, idxib, ai, aj, ee,
            rows0, rows1, zden, acc_sh, den_sh, sem0, sem1, semi):
        core = lax.axis_index("c")
        sid = lax.axis_index("s")
        wid = sid * NC + core
        zero16 = jnp.zeros((_LANES,), jnp.float32)

        def zb_body(i, _):
            for kk in range(D // _LANES):
                rows0[i, pl.ds(kk * _LANES, _LANES)] = zero16
            return 0
        lax.fori_loop(0, _W, zb_body, 0)

        def zd_body(i, _):
            zden[pl.ds(i * _LANES, _LANES)] = zero16
            return 0
        lax.fori_loop(0, zrows // _LANES, zd_body, 0)

        for j in range(nz):
            pltpu.sync_copy(rows0, acc_sh.at[pl.ds(sid * zrows + j * _W, _W)])
        pltpu.sync_copy(zden, den_sh.at[pl.ds(sid * zrows, zrows)])
        plsc.subcore_barrier()

        def stage_idx(w, b):
            pltpu.sync_copy(e3_h.at[wid, w], e3buf)
            for v in range(_W // _LANES):
                sl = pl.ds(v * _LANES, _LANES)
                t16 = e3buf[2, sl] * Np
                d16 = e3buf[1, sl]
                gidxb[b, sl] = t16 + e3buf[0, sl]
                idxib[b, sl] = t16 + d16
                dstb[b, sl] = d16

        stage_idx(0, 0)
        pltpu.async_copy(xw_h.at[gidxb.at[0]], rows0, sem0)

        def pair_body(i, _):
            for b in range(2):
                w = 2 * i + b
                rbuf, rsem = (rows0, sem0) if b == 0 else (rows1, sem1)
                obuf, osem = (rows1, sem1) if b == 0 else (rows0, sem0)
                nb = 1 - b
                pltpu.async_copy(aq_h.at[idxib.at[b]], ai, semi).wait()
                pltpu.async_copy(ak_h.at[gidxb.at[b]], aj, semi).wait()
                for v in range(_W // _LANES):
                    sl = pl.ds(v * _LANES, _LANES)
                    e16 = ai[sl] + aj[sl]
                    e16 = jnp.where(e16 >= 0.0, e16, 0.2 * e16)
                    ee[sl] = jnp.exp(e16)
                pltpu.sync_copy(ee, den_sh.at[dstb.at[b]], add=True)
                pltpu.make_async_copy(xw_h.at[gidxb.at[b]], rbuf, rsem).wait()

                @pl.when(w + 1 < nwin)
                def _():
                    stage_idx(w + 1, nb)
                    pltpu.async_copy(xw_h.at[gidxb.at[nb]], obuf, osem)

                def scale_grp(g, _):
                    a16 = ee[pl.ds(g * _LANES, _LANES)]
                    for j in range(_LANES):
                        a = a16[j]
                        r = g * _LANES + j
                        for kk in range(D // _LANES):
                            sl = pl.ds(kk * _LANES, _LANES)
                            rbuf[r, sl] = rbuf[r, sl] * a
                    return 0
                lax.fori_loop(0, _W // _LANES, scale_grp, 0)
                pltpu.sync_copy(rbuf, acc_sh.at[dstb.at[b]], add=True)
            return 0

        lax.fori_loop(0, nwin // 2, pair_body, 0)
        plsc.subcore_barrier()

        @pl.when(sid == 0)
        def _():
            pltpu.sync_copy(acc_sh, acc_out.at[core])
            pltpu.sync_copy(den_sh, den_out.at[core])

    return run(edges3, aq_flat, ak_flat, xw_flat)


def _finish(acc, den3, b):
    _, Np, D = acc.shape
    NB = Np // _BN

    def body(a_ref, d_ref, b_ref, h_ref):
        a = a_ref[0] + a_ref[1]
        dd = d_ref[0] + d_ref[1] + 1e-16
        h_ref[...] = jnp.maximum(a / dd + b_ref[...], 0.0)

    return pl.pallas_call(
        body,
        grid=(NB,),
        in_specs=[
            pl.BlockSpec((2, _BN, D), lambda nb: (0, nb, 0)),
            pl.BlockSpec((2, _BN, 1), lambda nb: (0, nb, 0)),
            pl.BlockSpec((1, D), lambda nb: (0, 0)),
        ],
        out_specs=pl.BlockSpec((_BN, D), lambda nb: (nb, 0)),
        out_shape=jax.ShapeDtypeStruct((Np, D), jnp.float32),
    )(acc, den3, b)


def _head(acc, den3, b, Wl, bl):
    _, Np, D = acc.shape
    DO = Wl.shape[1]
    NB = Np // _BN

    def body(a_ref, d_ref, b_ref, wl_ref, bl_ref, o_ref):
        a = a_ref[0] + a_ref[1]
        dd = d_ref[0] + d_ref[1] + 1e-16
        h = jnp.maximum(a / dd + b_ref[...], 0.0)
        logits = jnp.dot(h, wl_ref[...], preferred_element_type=jnp.float32)
        logits = logits + bl_ref[...]
        m = jnp.max(logits, axis=1, keepdims=True)
        s = logits - m
        o_ref[...] = s - jnp.log(jnp.sum(jnp.exp(s), axis=1, keepdims=True))

    return pl.pallas_call(
        body,
        grid=(NB,),
        in_specs=[
            pl.BlockSpec((2, _BN, D), lambda nb: (0, nb, 0)),
            pl.BlockSpec((2, _BN, 1), lambda nb: (0, nb, 0)),
            pl.BlockSpec((1, D), lambda nb: (0, 0)),
            pl.BlockSpec((D, DO), lambda nb: (0, 0)),
            pl.BlockSpec((1, DO), lambda nb: (0, 0)),
        ],
        out_specs=pl.BlockSpec((_BN, DO), lambda nb: (nb, 0)),
        out_shape=jax.ShapeDtypeStruct((Np, DO), jnp.float32),
    )(acc, den3, b, Wl, bl)


def kernel(x, edge_index, edge_type, tensor_slice,
           W1, q1, k1, b1, W2, q2, k2, b2, Wl, bl):
    N, D = x.shape
    R = W1.shape[0]
    E = edge_type.shape[0]

    Np = ((N + _BN - 1) // _BN) * _BN
    NW = 32
    wchunk = 2 * _W
    ew = ((E + NW * wchunk - 1) // (NW * wchunk)) * wchunk
    Ep = ew * NW

    xp = jnp.pad(x, ((0, Np - N), (0, 0)))
    src = edge_index[0]
    dst = edge_index[1]
    pad = Ep - E
    src_p = jnp.concatenate([src, jnp.zeros((pad,), jnp.int32)])
    dst_p = jnp.concatenate(
        [dst, Np - 8 + (jnp.arange(pad, dtype=jnp.int32) % 8)])
    et_p = jnp.concatenate([edge_type, jnp.zeros((pad,), jnp.int32)])
    nwin = ew // _W
    edges3 = jnp.stack([src_p.reshape(NW, nwin, _W),
                        dst_p.reshape(NW, nwin, _W),
                        et_p.reshape(NW, nwin, _W)], axis=2)

    q1r = q1.reshape(R, 1, D)
    k1r = k1.reshape(R, 1, D)
    q2r = q2.reshape(R, 1, D)
    k2r = k2.reshape(R, 1, D)

    xw1, aq1, ak1 = _transform(xp, W1, q1r, k1r)
    acc1, den1 = _edge_pass(edges3, aq1.reshape(R * Np), ak1.reshape(R * Np),
                            xw1.reshape(R * Np, D), Np, D)
    h = _finish(acc1, den1.reshape(2, Np, 1), b1.reshape(1, D))

    xw2, aq2, ak2 = _transform(h, W2, q2r, k2r)
    acc2, den2 = _edge_pass(edges3, aq2.reshape(R * Np), ak2.reshape(R * Np),
                            xw2.reshape(R * Np, D), Np, D)
    out = _head(acc2, den2.reshape(2, Np, 1), b2.reshape(1, D),
                Wl, bl.reshape(1, Wl.shape[1]))
    return out[:N]
